# Initial kernel scaffold; baseline (speedup 1.0000x reference)
import jax
import jax.numpy as jnp
from jax.experimental import pallas as pl


def kernel(gt_matches0, gt_matches1, scores):
    raise NotImplementedError("write your pallas kernel here")


# speedup vs baseline: 209.9067x; 209.9067x over previous
import jax
import jax.numpy as jnp
from jax.experimental import pallas as pl
from jax.experimental.pallas import tpu as pltpu

_B, _N, _M = 16, 2048, 2048
_GAMMA = 0.5
_RB = 256
_NBLK = _N // _RB
_NEG = float("-inf")


def _triplet_body(scores_ref, g0_ref, g1_ref, lc_ref, lr_ref, out_ref,
                  cv1_ref, cv2_ref, ca1_ref, cpos_ref, acc_ref):
    b = pl.program_id(0)
    rb = pl.program_id(1)
    s = scores_ref[0]
    g0 = g0_ref[0, 0]
    g1 = g1_ref[0]
    lc = lc_ref[0, 0]
    lr = lr_ref[0]

    @pl.when(jnp.logical_and(b == 0, rb == 0))
    def _():
        acc_ref[0, 0] = 0.0

    @pl.when(rb == 0)
    def _():
        cv1_ref[...] = jnp.full((1, _M), _NEG, jnp.float32)
        cv2_ref[...] = jnp.full((1, _M), _NEG, jnp.float32)
        ca1_ref[...] = jnp.zeros((1, _M), jnp.int32)
        cpos_ref[...] = jnp.zeros((1, _M), jnp.float32)

    col_idx = jax.lax.broadcasted_iota(jnp.int32, (_RB, _M), 1)
    row_idx = jax.lax.broadcasted_iota(jnp.int32, (_RB, _M), 0) + rb * _RB

    rm1 = jnp.max(s, axis=1, keepdims=True)
    ra1 = jnp.min(jnp.where(s == rm1, col_idx, _M), axis=1, keepdims=True)
    rm2 = jnp.max(jnp.where(col_idx == ra1, _NEG, s), axis=1, keepdims=True)
    rpos = jnp.sum(jnp.where(col_idx == g0, s, 0.0), axis=1, keepdims=True)
    rpos = jnp.where(g0 == _M, lc, rpos)
    take = lc > rm1
    v1 = jnp.where(take, lc, rm1)
    a1 = jnp.where(take, _M, ra1)
    v2 = jnp.where(take, rm1, jnp.maximum(rm2, lc))
    neg = jnp.where(a1 == g0, v2, v1)
    acc_ref[0, 0] += jnp.sum(jnp.maximum(neg - rpos + _GAMMA, 0.0))

    cm1 = jnp.max(s, axis=0, keepdims=True)
    ca1b = jnp.min(jnp.where(s == cm1, row_idx, _N), axis=0, keepdims=True)
    cm2 = jnp.max(jnp.where(row_idx == ca1b, _NEG, s), axis=0, keepdims=True)
    cpos_ref[...] += jnp.sum(jnp.where(row_idx == g1, s, 0.0), axis=0,
                             keepdims=True)
    pv1, pv2, pa1 = cv1_ref[...], cv2_ref[...], ca1_ref[...]
    takec = cm1 > pv1
    cv1_ref[...] = jnp.where(takec, cm1, pv1)
    ca1_ref[...] = jnp.where(takec, ca1b, pa1)
    cv2_ref[...] = jnp.where(takec, jnp.maximum(pv1, cm2),
                             jnp.maximum(pv2, cm1))

    @pl.when(rb == _NBLK - 1)
    def _():
        pv1, pv2, pa1 = cv1_ref[...], cv2_ref[...], ca1_ref[...]
        takeb = lr > pv1
        fv1 = jnp.where(takeb, lr, pv1)
        fa1 = jnp.where(takeb, _N, pa1)
        fv2 = jnp.where(takeb, pv1, jnp.maximum(pv2, lr))
        fpos = jnp.where(g1 == _N, lr, cpos_ref[...])
        fneg = jnp.where(fa1 == g1, fv2, fv1)
        acc_ref[0, 0] += jnp.sum(jnp.maximum(fneg - fpos + _GAMMA, 0.0))

    out_ref[...] = jnp.full((1, 1), acc_ref[0, 0] * (1.0 / (2 * _B * _N)),
                            jnp.float32)


def _run(scores, g0r, g1r, lc, lr):
    return pl.pallas_call(
        _triplet_body,
        grid=(_B, _NBLK),
        in_specs=[
            pl.BlockSpec((1, _RB, _M), lambda b, rb: (b, rb, 0)),
            pl.BlockSpec((1, 1, _RB, 1), lambda b, rb: (b, rb, 0, 0)),
            pl.BlockSpec((1, 1, _M), lambda b, rb: (b, 0, 0)),
            pl.BlockSpec((1, 1, _RB, 1), lambda b, rb: (b, rb, 0, 0)),
            pl.BlockSpec((1, 1, _M), lambda b, rb: (b, 0, 0)),
        ],
        out_specs=pl.BlockSpec((1, 1), lambda b, rb: (0, 0)),
        out_shape=jax.ShapeDtypeStruct((1, 1), jnp.float32),
        scratch_shapes=[
            pltpu.VMEM((1, _M), jnp.float32),
            pltpu.VMEM((1, _M), jnp.float32),
            pltpu.VMEM((1, _M), jnp.int32),
            pltpu.VMEM((1, _M), jnp.float32),
            pltpu.SMEM((1, 1), jnp.float32),
        ],
    )(scores, g0r, g1r, lc, lr)


def kernel(gt_matches0, gt_matches1, scores):
    g0 = jnp.where(gt_matches0 == -1, _M, gt_matches0).astype(jnp.int32)
    g1 = jnp.where(gt_matches1 == -1, _N, gt_matches1).astype(jnp.int32)
    g0r = g0.reshape(_B, _NBLK, _RB, 1)
    g1r = g1.reshape(_B, 1, _M)
    lc = scores[:, :_N, _M].reshape(_B, _NBLK, _RB, 1)
    lr = scores[:, _N, :_M].reshape(_B, 1, _M)
    out = _run(scores, g0r, g1r, lc, lr)
    return out[0, 0]


# speedup vs baseline: 246.3234x; 1.1735x over previous
import jax
import jax.numpy as jnp
from jax.experimental import pallas as pl
from jax.experimental.pallas import tpu as pltpu

_B, _N, _M = 16, 2048, 2048
_GAMMA = 0.5
_RB = 256
_NBLK = _N // _RB
_NEG = float("-inf")


def _triplet_body(scores_ref, g0_ref, g1_ref, lc_ref, lr_ref, out_ref,
                  cv1_ref, cv2_ref, cpos_ref, acc_ref):
    b = pl.program_id(0)
    rb = pl.program_id(1)
    s = scores_ref[0]
    g0 = g0_ref[0, 0]
    g1 = g1_ref[0]
    lc = lc_ref[0, 0]
    lr = lr_ref[0]

    @pl.when(jnp.logical_and(b == 0, rb == 0))
    def _():
        acc_ref[0, 0] = 0.0

    @pl.when(rb == 0)
    def _():
        cv1_ref[...] = jnp.full((1, _M), _NEG, jnp.float32)
        cv2_ref[...] = jnp.full((1, _M), _NEG, jnp.float32)
        cpos_ref[...] = jnp.zeros((1, _M), jnp.float32)

    col_idx = jax.lax.broadcasted_iota(jnp.int32, (_RB, _M), 1)
    row_idx = jax.lax.broadcasted_iota(jnp.int32, (_RB, _M), 0)

    rm1 = jnp.max(s, axis=1, keepdims=True)
    rm2 = jnp.max(jnp.where(s == rm1, _NEG, s), axis=1, keepdims=True)
    rpos = jnp.sum(jnp.where(col_idx == g0, s, 0.0), axis=1, keepdims=True)
    rpos = jnp.where(g0 == _M, lc, rpos)
    fv1 = jnp.maximum(rm1, lc)
    fv2 = jnp.maximum(jnp.minimum(rm1, lc), rm2)
    neg = jnp.where(rpos == fv1, fv2, fv1)
    acc_ref[0, 0] += jnp.sum(jnp.maximum(neg - rpos + _GAMMA, 0.0))

    cm1 = jnp.max(s, axis=0, keepdims=True)
    cm2 = jnp.max(jnp.where(s == cm1, _NEG, s), axis=0, keepdims=True)
    cpos_ref[...] += jnp.sum(jnp.where(row_idx == (g1 - rb * _RB), s, 0.0),
                             axis=0, keepdims=True)
    pv1, pv2 = cv1_ref[...], cv2_ref[...]
    cv1_ref[...] = jnp.maximum(pv1, cm1)
    cv2_ref[...] = jnp.maximum(jnp.maximum(pv2, cm2), jnp.minimum(pv1, cm1))

    @pl.when(rb == _NBLK - 1)
    def _():
        pv1, pv2 = cv1_ref[...], cv2_ref[...]
        fv1 = jnp.maximum(pv1, lr)
        fv2 = jnp.maximum(jnp.minimum(pv1, lr), pv2)
        fpos = jnp.where(g1 == _N, lr, cpos_ref[...])
        fneg = jnp.where(fpos == fv1, fv2, fv1)
        acc_ref[0, 0] += jnp.sum(jnp.maximum(fneg - fpos + _GAMMA, 0.0))

    out_ref[...] = jnp.full((1, 1), acc_ref[0, 0] * (1.0 / (2 * _B * _N)),
                            jnp.float32)


def _run(scores, g0r, g1r, lc, lr):
    return pl.pallas_call(
        _triplet_body,
        grid=(_B, _NBLK),
        in_specs=[
            pl.BlockSpec((1, _RB, _M), lambda b, rb: (b, rb, 0)),
            pl.BlockSpec((1, 1, _RB, 1), lambda b, rb: (b, rb, 0, 0)),
            pl.BlockSpec((1, 1, _M), lambda b, rb: (b, 0, 0)),
            pl.BlockSpec((1, 1, _RB, 1), lambda b, rb: (b, rb, 0, 0)),
            pl.BlockSpec((1, 1, _M), lambda b, rb: (b, 0, 0)),
        ],
        out_specs=pl.BlockSpec((1, 1), lambda b, rb: (0, 0)),
        out_shape=jax.ShapeDtypeStruct((1, 1), jnp.float32),
        scratch_shapes=[
            pltpu.VMEM((1, _M), jnp.float32),
            pltpu.VMEM((1, _M), jnp.float32),
            pltpu.VMEM((1, _M), jnp.float32),
            pltpu.SMEM((1, 1), jnp.float32),
        ],
    )(scores, g0r, g1r, lc, lr)


def kernel(gt_matches0, gt_matches1, scores):
    g0 = jnp.where(gt_matches0 == -1, _M, gt_matches0).astype(jnp.int32)
    g1 = jnp.where(gt_matches1 == -1, _N, gt_matches1).astype(jnp.int32)
    g0r = g0.reshape(_B, _NBLK, _RB, 1)
    g1r = g1.reshape(_B, 1, _M)
    lc = scores[:, :_N, _M].reshape(_B, _NBLK, _RB, 1)
    lr = scores[:, _N, :_M].reshape(_B, 1, _M)
    out = _run(scores, g0r, g1r, lc, lr)
    return out[0, 0]


# speedup vs baseline: 260.8285x; 1.0589x over previous
import jax
import jax.numpy as jnp
from jax.experimental import pallas as pl
from jax.experimental.pallas import tpu as pltpu

_B, _N, _M = 16, 2048, 2048
_GAMMA = 0.5
_RB = 512
_NBLK = _N // _RB
_NEG = float("-inf")


def _triplet_body(scores_ref, g0_ref, g1_ref, lc_ref, lr_ref, out_ref,
                  cv1_ref, cv2_ref, cpos_ref, acc_ref):
    b = pl.program_id(0)
    rb = pl.program_id(1)
    s = scores_ref[0]
    g0 = g0_ref[0, 0]
    g1 = g1_ref[0]
    lc = lc_ref[0, 0]
    lr = lr_ref[0]

    @pl.when(jnp.logical_and(b == 0, rb == 0))
    def _():
        acc_ref[0, 0] = 0.0

    @pl.when(rb == 0)
    def _():
        cv1_ref[...] = jnp.full((1, _M), _NEG, jnp.float32)
        cv2_ref[...] = jnp.full((1, _M), _NEG, jnp.float32)
        cpos_ref[...] = jnp.zeros((1, _M), jnp.float32)

    col_idx = jax.lax.broadcasted_iota(jnp.int32, (_RB, _M), 1)
    row_idx = jax.lax.broadcasted_iota(jnp.int32, (_RB, _M), 0)

    rm1 = jnp.max(s, axis=1, keepdims=True)
    rm2 = jnp.max(jnp.where(s == rm1, _NEG, s), axis=1, keepdims=True)
    rpos = jnp.sum(jnp.where(col_idx == g0, s, 0.0), axis=1, keepdims=True)
    rpos = jnp.where(g0 == _M, lc, rpos)
    fv1 = jnp.maximum(rm1, lc)
    fv2 = jnp.maximum(jnp.minimum(rm1, lc), rm2)
    neg = jnp.where(rpos == fv1, fv2, fv1)
    acc_ref[0, 0] += jnp.sum(jnp.maximum(neg - rpos + _GAMMA, 0.0))

    cm1 = jnp.max(s, axis=0, keepdims=True)
    cm2 = jnp.max(jnp.where(s == cm1, _NEG, s), axis=0, keepdims=True)
    cpos_ref[...] += jnp.sum(jnp.where(row_idx == (g1 - rb * _RB), s, 0.0),
                             axis=0, keepdims=True)
    pv1, pv2 = cv1_ref[...], cv2_ref[...]
    cv1_ref[...] = jnp.maximum(pv1, cm1)
    cv2_ref[...] = jnp.maximum(jnp.maximum(pv2, cm2), jnp.minimum(pv1, cm1))

    @pl.when(rb == _NBLK - 1)
    def _():
        pv1, pv2 = cv1_ref[...], cv2_ref[...]
        fv1 = jnp.maximum(pv1, lr)
        fv2 = jnp.maximum(jnp.minimum(pv1, lr), pv2)
        fpos = jnp.where(g1 == _N, lr, cpos_ref[...])
        fneg = jnp.where(fpos == fv1, fv2, fv1)
        acc_ref[0, 0] += jnp.sum(jnp.maximum(fneg - fpos + _GAMMA, 0.0))

    out_ref[...] = jnp.full((1, 1), acc_ref[0, 0] * (1.0 / (2 * _B * _N)),
                            jnp.float32)


def _run(scores, g0r, g1r, lc, lr):
    return pl.pallas_call(
        _triplet_body,
        grid=(_B, _NBLK),
        in_specs=[
            pl.BlockSpec((1, _RB, _M), lambda b, rb: (b, rb, 0)),
            pl.BlockSpec((1, 1, _RB, 1), lambda b, rb: (b, rb, 0, 0)),
            pl.BlockSpec((1, 1, _M), lambda b, rb: (b, 0, 0)),
            pl.BlockSpec((1, 1, _RB, 1), lambda b, rb: (b, rb, 0, 0)),
            pl.BlockSpec((1, 1, _M), lambda b, rb: (b, 0, 0)),
        ],
        out_specs=pl.BlockSpec((1, 1), lambda b, rb: (0, 0)),
        out_shape=jax.ShapeDtypeStruct((1, 1), jnp.float32),
        scratch_shapes=[
            pltpu.VMEM((1, _M), jnp.float32),
            pltpu.VMEM((1, _M), jnp.float32),
            pltpu.VMEM((1, _M), jnp.float32),
            pltpu.SMEM((1, 1), jnp.float32),
        ],
    )(scores, g0r, g1r, lc, lr)


def kernel(gt_matches0, gt_matches1, scores):
    g0 = jnp.where(gt_matches0 == -1, _M, gt_matches0).astype(jnp.int32)
    g1 = jnp.where(gt_matches1 == -1, _N, gt_matches1).astype(jnp.int32)
    g0r = g0.reshape(_B, _NBLK, _RB, 1)
    g1r = g1.reshape(_B, 1, _M)
    lc = scores[:, :_N, _M].reshape(_B, _NBLK, _RB, 1)
    lr = scores[:, _N, :_M].reshape(_B, 1, _M)
    out = _run(scores, g0r, g1r, lc, lr)
    return out[0, 0]


# speedup vs baseline: 319.8956x; 1.2265x over previous
import jax
import jax.numpy as jnp
from jax.experimental import pallas as pl
from jax.experimental.pallas import tpu as pltpu

_B, _N, _M = 16, 2048, 2048
_GAMMA = 0.5
_NEG = float("-inf")


def _triplet_body(scores_ref, g0_ref, g1_ref, out_ref, acc_ref):
    b = pl.program_id(0)
    s = scores_ref[0]
    g0 = g0_ref[0]
    g1 = g1_ref[0]

    @pl.when(b == 0)
    def _():
        acc_ref[0, 0] = 0.0

    sr = s[:_N, :]
    col_idx = jax.lax.broadcasted_iota(jnp.int32, (_N, _M + 1), 1)
    rm1 = jnp.max(sr, axis=1, keepdims=True)
    rm2 = jnp.max(jnp.where(sr == rm1, _NEG, sr), axis=1, keepdims=True)
    rpos = jnp.sum(jnp.where(col_idx == g0, sr, 0.0), axis=1, keepdims=True)
    neg = jnp.where(rpos == rm1, rm2, rm1)
    acc_ref[0, 0] += jnp.sum(jnp.maximum(neg - rpos + _GAMMA, 0.0))

    sc = s[:, :_M]
    row_idx = jax.lax.broadcasted_iota(jnp.int32, (_N + 1, _M), 0)
    cm1 = jnp.max(sc, axis=0, keepdims=True)
    cm2 = jnp.max(jnp.where(sc == cm1, _NEG, sc), axis=0, keepdims=True)
    cpos = jnp.sum(jnp.where(row_idx == g1, sc, 0.0), axis=0, keepdims=True)
    cneg = jnp.where(cpos == cm1, cm2, cm1)
    acc_ref[0, 0] += jnp.sum(jnp.maximum(cneg - cpos + _GAMMA, 0.0))

    out_ref[...] = jnp.full((1, 1), acc_ref[0, 0] * (1.0 / (2 * _B * _N)),
                            jnp.float32)


def _run(scores, g0r, g1r):
    return pl.pallas_call(
        _triplet_body,
        grid=(_B,),
        in_specs=[
            pl.BlockSpec((1, _N + 1, _M + 1), lambda b: (b, 0, 0)),
            pl.BlockSpec((1, _N, 1), lambda b: (b, 0, 0)),
            pl.BlockSpec((1, 1, _M), lambda b: (b, 0, 0)),
        ],
        out_specs=pl.BlockSpec((1, 1), lambda b: (0, 0)),
        out_shape=jax.ShapeDtypeStruct((1, 1), jnp.float32),
        scratch_shapes=[
            pltpu.SMEM((1, 1), jnp.float32),
        ],
    )(scores, g0r, g1r)


def kernel(gt_matches0, gt_matches1, scores):
    g0 = jnp.where(gt_matches0 == -1, _M, gt_matches0).astype(jnp.int32)
    g1 = jnp.where(gt_matches1 == -1, _N, gt_matches1).astype(jnp.int32)
    g0r = g0.reshape(_B, _N, 1)
    g1r = g1.reshape(_B, 1, _M)
    out = _run(scores, g0r, g1r)
    return out[0, 0]
